# baseline (speedup 1.0000x reference)
import functools

import jax
import jax.numpy as jnp
from jax import lax
from jax.experimental import pallas as pl
from jax.experimental.pallas import tpu as pltpu
from jax.experimental.pallas import tpu_sc as plsc

N_ROWS = 320000
N_SEG = 10000
D = 128
S_PAD = 10240
H_SC = 185856
C_ROWS = 128
N_CHUNKS = H_SC // C_ROWS
NW = 32
MAX_ORD = -(-N_CHUNKS // NW)
N_STEPS = -(-MAX_ORD // 2)
STRIPE = S_PAD // 16


def _sc_body(vals, seg_ids, acc_out, cnt_out,
             acc_sp, cnt_sp, rows0, rows1, ids0, ids1, ones_v, zcnt_v,
             semr0, semr1, semi0, semi1, semc0, semc1):
    c = lax.axis_index("c")
    s = lax.axis_index("s")
    w = s * 2 + c
    rows = (rows0, rows1)
    ids = (ids0, ids1)
    semr = (semr0, semr1)
    semi = (semi0, semi1)
    semc = (semc0, semc1)

    z16 = jnp.zeros((16,), jnp.float32)
    o16 = jnp.ones((16,), jnp.float32)

    def _zrow(i, carry):
        rows0[i // 8, pl.ds((i % 8) * 16, 16)] = z16
        return carry
    lax.fori_loop(0, C_ROWS * 8, _zrow, 0)

    def _zcnt(i, carry):
        zcnt_v[pl.ds(i * 16, 16)] = z16
        return carry
    lax.fori_loop(0, STRIPE // 16, _zcnt, 0)

    for t in range(8):
        ones_v[pl.ds(t * 16, 16)] = o16

    base = s * STRIPE
    for q in range(STRIPE // C_ROWS):
        pltpu.sync_copy(rows0, acc_sp.at[pl.ds(base + q * C_ROWS, C_ROWS), :])
    pltpu.sync_copy(zcnt_v, cnt_sp.at[pl.ds(base, STRIPE)])
    plsc.subcore_barrier()

    def _issue(k, b):
        @pl.when(k < N_CHUNKS)
        def _():
            pltpu.async_copy(vals.at[pl.ds(k * C_ROWS, C_ROWS), :],
                             rows[b], semr[b])
            pltpu.async_copy(seg_ids.at[pl.ds(k * C_ROWS, C_ROWS)],
                             ids[b], semi[b])

    _issue(w, 0)

    def _step(i, carry):
        for b in range(2):
            ordinal = 2 * i + b
            k = w + ordinal * NW

            @pl.when(k < N_CHUNKS)
            def _():
                @pl.when(ordinal >= 1)
                def _():
                    pltpu.make_async_copy(ones_v, cnt_sp.at[ids[1 - b]],
                                          semc[1 - b]).wait()
                _issue(k + NW, 1 - b)
                pltpu.make_async_copy(vals.at[pl.ds(k * C_ROWS, C_ROWS), :],
                                      rows[b], semr[b]).wait()
                pltpu.make_async_copy(seg_ids.at[pl.ds(k * C_ROWS, C_ROWS)],
                                      ids[b], semi[b]).wait()
                pltpu.sync_copy(rows[b], acc_sp.at[ids[b]], add=True)
                pltpu.async_copy(ones_v, cnt_sp.at[ids[b]], semc[b], add=True)
        return carry
    lax.fori_loop(0, N_STEPS, _step, 0)

    n_valid = (N_CHUNKS - w + NW - 1) // NW
    last_b = (n_valid - 1) % 2

    @pl.when(last_b == 0)
    def _():
        pltpu.make_async_copy(ones_v, cnt_sp.at[ids0], semc0).wait()

    @pl.when(last_b == 1)
    def _():
        pltpu.make_async_copy(ones_v, cnt_sp.at[ids1], semc1).wait()

    plsc.subcore_barrier()

    pltpu.sync_copy(acc_sp.at[pl.ds(base, STRIPE), :],
                    acc_out.at[c, pl.ds(base, STRIPE), :])
    pltpu.sync_copy(cnt_sp.at[pl.ds(base, STRIPE)],
                    cnt_out.at[c, pl.ds(base, STRIPE)])


_sc_agg = functools.partial(
    pl.kernel,
    out_type=[
        jax.ShapeDtypeStruct((2, S_PAD, D), jnp.float32),
        jax.ShapeDtypeStruct((2, S_PAD), jnp.float32),
    ],
    mesh=plsc.VectorSubcoreMesh(core_axis_name="c", subcore_axis_name="s"),
    scratch_types=[
        pltpu.VMEM_SHARED((S_PAD, D), jnp.float32),
        pltpu.VMEM_SHARED((S_PAD,), jnp.float32),
        pltpu.VMEM((C_ROWS, D), jnp.float32),
        pltpu.VMEM((C_ROWS, D), jnp.float32),
        pltpu.VMEM((C_ROWS,), jnp.int32),
        pltpu.VMEM((C_ROWS,), jnp.int32),
        pltpu.VMEM((C_ROWS,), jnp.float32),
        pltpu.VMEM((STRIPE,), jnp.float32),
        pltpu.SemaphoreType.DMA,
        pltpu.SemaphoreType.DMA,
        pltpu.SemaphoreType.DMA,
        pltpu.SemaphoreType.DMA,
        pltpu.SemaphoreType.DMA,
        pltpu.SemaphoreType.DMA,
    ],
)(_sc_body)


SEG_T = 128
N_TILES = S_PAD // SEG_T
TB = 1024
NB_TC = (N_ROWS - H_SC) // TB
NBUF = 4


def _tc_body(meta_ref, vals, seg_ids, acc_ref, cnt_ref, rows, ids, semr, semi):
    t = pl.program_id(0)
    blo = meta_ref[t]
    nblk = meta_ref[N_TILES + t] - blo

    acc_ref[...] = jnp.zeros((SEG_T, D), jnp.float32)
    cnt_ref[...] = jnp.zeros((SEG_T, 1), jnp.float32)

    def _tissue(j, b):
        @pl.when(j < nblk)
        def _():
            off = H_SC + (blo + j) * TB
            pltpu.make_async_copy(vals.at[pl.ds(off, TB), :],
                                  rows.at[b], semr.at[b]).start()
            pltpu.make_async_copy(seg_ids.at[pl.ds(off, TB)],
                                  ids.at[b], semi.at[b]).start()

    for b in range(NBUF - 1):
        _tissue(b, b)
    seg_iota = t * SEG_T + lax.broadcasted_iota(jnp.int32, (SEG_T, TB), 0)

    def _tstep(i, carry):
        for b in range(NBUF):
            j = NBUF * i + b

            @pl.when(j < nblk)
            def _():
                _tissue(j + NBUF - 1, (b + NBUF - 1) % NBUF)
                off = H_SC + (blo + j) * TB
                pltpu.make_async_copy(vals.at[pl.ds(off, TB), :],
                                      rows.at[b], semr.at[b]).wait()
                pltpu.make_async_copy(seg_ids.at[pl.ds(off, TB)],
                                      ids.at[b], semi.at[b]).wait()
                ids_blk = ids[b].reshape(1, TB)
                onehot = (seg_iota == ids_blk).astype(jnp.float32)
                r = rows[b]
                r_hi = r.astype(jnp.bfloat16).astype(jnp.float32)
                r_cat = jnp.concatenate([r_hi, r - r_hi], axis=1)
                dims = (((1,), (0,)), ((), ()))
                prod = lax.dot_general(onehot, r_cat, dims,
                                       preferred_element_type=jnp.float32)
                acc_ref[...] += prod[:, :D] + prod[:, D:]
                cnt_ref[...] += jnp.sum(onehot, axis=1, keepdims=True)
        return carry
    lax.fori_loop(0, (nblk + NBUF - 1) // NBUF, _tstep, 0)


_tc_part = pl.pallas_call(
    _tc_body,
    grid_spec=pltpu.PrefetchScalarGridSpec(
        num_scalar_prefetch=1,
        grid=(N_TILES,),
        in_specs=[
            pl.BlockSpec(memory_space=pl.ANY),
            pl.BlockSpec(memory_space=pl.ANY),
        ],
        out_specs=[
            pl.BlockSpec((SEG_T, D), lambda t, b_ref: (t, 0)),
            pl.BlockSpec((SEG_T, 1), lambda t, b_ref: (t, 0)),
        ],
        scratch_shapes=[
            pltpu.VMEM((NBUF, TB, D), jnp.float32),
            pltpu.VMEM((NBUF, TB), jnp.int32),
            pltpu.SemaphoreType.DMA((NBUF,)),
            pltpu.SemaphoreType.DMA((NBUF,)),
        ],
    ),
    out_shape=[
        jax.ShapeDtypeStruct((S_PAD, D), jnp.float32),
        jax.ShapeDtypeStruct((S_PAD, 1), jnp.float32),
    ],
)


RB = 2000


def _merge_body(a_ref, c_ref, at_ref, ct_ref, o_ref):
    sums = a_ref[0] + a_ref[1] + at_ref[...]
    cnts = c_ref[0] + c_ref[1] + ct_ref[...]
    o_ref[...] = sums / jnp.maximum(cnts, 1.0)


_merge = pl.pallas_call(
    _merge_body,
    grid=(N_SEG // RB,),
    in_specs=[
        pl.BlockSpec((2, RB, D), lambda r: (0, r, 0)),
        pl.BlockSpec((2, RB, 1), lambda r: (0, r, 0)),
        pl.BlockSpec((RB, D), lambda r: (r, 0)),
        pl.BlockSpec((RB, 1), lambda r: (r, 0)),
    ],
    out_specs=pl.BlockSpec((RB, D), lambda r: (r, 0)),
    out_shape=jax.ShapeDtypeStruct((N_SEG, D), jnp.float32),
)


def kernel(input_values, segment_ids):
    ids32 = segment_ids.astype(jnp.int32)
    first = lax.slice(ids32, (H_SC,), (N_ROWS,), (TB,))
    nxt = jnp.concatenate(
        [first[1:], jnp.full((1,), jnp.iinfo(jnp.int32).max, jnp.int32)])
    tile_lo = jnp.arange(N_TILES, dtype=jnp.int32) * SEG_T
    blo = jnp.sum(nxt[None, :] < tile_lo[:, None], axis=1,
                  dtype=jnp.int32)
    bhi = jnp.sum(first[None, :] < (tile_lo[:, None] + SEG_T), axis=1,
                  dtype=jnp.int32)
    meta = jnp.concatenate([blo, jnp.maximum(bhi, blo)])
    acc_sc, cnt_sc = _sc_agg(input_values, ids32)
    acc_tc, cnt_tc = _tc_part(meta, input_values, ids32)
    return _merge(acc_sc, cnt_sc.reshape(2, S_PAD, 1), acc_tc, cnt_tc)


# speedup vs baseline: 1.1774x; 1.1774x over previous
import functools

import jax
import jax.numpy as jnp
from jax import lax
from jax.experimental import pallas as pl
from jax.experimental.pallas import tpu as pltpu
from jax.experimental.pallas import tpu_sc as plsc

N_ROWS = 320000
N_SEG = 10000
D = 128
S_PAD = 10240
H_SC = 209408
C_ROWS = 128
N_CHUNKS = H_SC // C_ROWS
NW = 32
MAX_ORD = -(-N_CHUNKS // NW)
N_STEPS = -(-MAX_ORD // 2)
STRIPE = S_PAD // 16


def _sc_body(vals, seg_ids, acc_out, cnt_out,
             acc_sp, cnt_sp, rows0, rows1, ids0, ids1, ones_v, zcnt_v,
             semr0, semr1, semi0, semi1, semc0, semc1):
    c = lax.axis_index("c")
    s = lax.axis_index("s")
    w = s * 2 + c
    rows = (rows0, rows1)
    ids = (ids0, ids1)
    semr = (semr0, semr1)
    semi = (semi0, semi1)
    semc = (semc0, semc1)

    z16 = jnp.zeros((16,), jnp.float32)
    o16 = jnp.ones((16,), jnp.float32)

    def _zrow(i, carry):
        rows0[i // 8, pl.ds((i % 8) * 16, 16)] = z16
        return carry
    lax.fori_loop(0, C_ROWS * 8, _zrow, 0)

    def _zcnt(i, carry):
        zcnt_v[pl.ds(i * 16, 16)] = z16
        return carry
    lax.fori_loop(0, STRIPE // 16, _zcnt, 0)

    for t in range(8):
        ones_v[pl.ds(t * 16, 16)] = o16

    base = s * STRIPE
    for q in range(STRIPE // C_ROWS):
        pltpu.sync_copy(rows0, acc_sp.at[pl.ds(base + q * C_ROWS, C_ROWS), :])
    pltpu.sync_copy(zcnt_v, cnt_sp.at[pl.ds(base, STRIPE)])
    plsc.subcore_barrier()

    def _issue(k, b):
        @pl.when(k < N_CHUNKS)
        def _():
            pltpu.async_copy(vals.at[pl.ds(k * C_ROWS, C_ROWS), :],
                             rows[b], semr[b])
            pltpu.async_copy(seg_ids.at[pl.ds(k * C_ROWS, C_ROWS)],
                             ids[b], semi[b])

    _issue(w, 0)

    def _step(i, carry):
        for b in range(2):
            ordinal = 2 * i + b
            k = w + ordinal * NW

            @pl.when(k < N_CHUNKS)
            def _():
                @pl.when(ordinal >= 1)
                def _():
                    pltpu.make_async_copy(ones_v, cnt_sp.at[ids[1 - b]],
                                          semc[1 - b]).wait()
                _issue(k + NW, 1 - b)
                pltpu.make_async_copy(vals.at[pl.ds(k * C_ROWS, C_ROWS), :],
                                      rows[b], semr[b]).wait()
                pltpu.make_async_copy(seg_ids.at[pl.ds(k * C_ROWS, C_ROWS)],
                                      ids[b], semi[b]).wait()
                pltpu.sync_copy(rows[b], acc_sp.at[ids[b]], add=True)
                pltpu.async_copy(ones_v, cnt_sp.at[ids[b]], semc[b], add=True)
        return carry
    lax.fori_loop(0, N_STEPS, _step, 0)

    n_valid = (N_CHUNKS - w + NW - 1) // NW
    last_b = (n_valid - 1) % 2

    @pl.when(last_b == 0)
    def _():
        pltpu.make_async_copy(ones_v, cnt_sp.at[ids0], semc0).wait()

    @pl.when(last_b == 1)
    def _():
        pltpu.make_async_copy(ones_v, cnt_sp.at[ids1], semc1).wait()

    plsc.subcore_barrier()

    pltpu.sync_copy(acc_sp.at[pl.ds(base, STRIPE), :],
                    acc_out.at[c, pl.ds(base, STRIPE), :])
    pltpu.sync_copy(cnt_sp.at[pl.ds(base, STRIPE)],
                    cnt_out.at[c, pl.ds(base, STRIPE)])


_sc_agg = functools.partial(
    pl.kernel,
    out_type=[
        jax.ShapeDtypeStruct((2, S_PAD, D), jnp.float32),
        jax.ShapeDtypeStruct((2, S_PAD), jnp.float32),
    ],
    mesh=plsc.VectorSubcoreMesh(core_axis_name="c", subcore_axis_name="s"),
    scratch_types=[
        pltpu.VMEM_SHARED((S_PAD, D), jnp.float32),
        pltpu.VMEM_SHARED((S_PAD,), jnp.float32),
        pltpu.VMEM((C_ROWS, D), jnp.float32),
        pltpu.VMEM((C_ROWS, D), jnp.float32),
        pltpu.VMEM((C_ROWS,), jnp.int32),
        pltpu.VMEM((C_ROWS,), jnp.int32),
        pltpu.VMEM((C_ROWS,), jnp.float32),
        pltpu.VMEM((STRIPE,), jnp.float32),
        pltpu.SemaphoreType.DMA,
        pltpu.SemaphoreType.DMA,
        pltpu.SemaphoreType.DMA,
        pltpu.SemaphoreType.DMA,
        pltpu.SemaphoreType.DMA,
        pltpu.SemaphoreType.DMA,
    ],
)(_sc_body)


SEG_T = 128
N_TILES = S_PAD // SEG_T
TB = 2048
NB_TC = (N_ROWS - H_SC) // TB
NBUF = 4


def _tc_body(meta_ref, vals, seg_ids, acc_ref, cnt_ref, rows, ids, semr, semi):
    t = pl.program_id(0)
    blo = meta_ref[t]
    nblk = meta_ref[N_TILES + t] - blo

    acc_ref[...] = jnp.zeros((SEG_T, D), jnp.float32)
    cnt_ref[...] = jnp.zeros((SEG_T, 1), jnp.float32)

    def _tissue(j, b):
        @pl.when(j < nblk)
        def _():
            off = H_SC + (blo + j) * TB
            pltpu.make_async_copy(vals.at[pl.ds(off, TB), :],
                                  rows.at[b], semr.at[b]).start()
            pltpu.make_async_copy(seg_ids.at[pl.ds(off, TB)],
                                  ids.at[b], semi.at[b]).start()

    for b in range(NBUF - 1):
        _tissue(b, b)
    seg_iota = t * SEG_T + lax.broadcasted_iota(jnp.int32, (SEG_T, TB), 0)

    def _tstep(i, carry):
        for b in range(NBUF):
            j = NBUF * i + b

            @pl.when(j < nblk)
            def _():
                _tissue(j + NBUF - 1, (b + NBUF - 1) % NBUF)
                off = H_SC + (blo + j) * TB
                pltpu.make_async_copy(vals.at[pl.ds(off, TB), :],
                                      rows.at[b], semr.at[b]).wait()
                pltpu.make_async_copy(seg_ids.at[pl.ds(off, TB)],
                                      ids.at[b], semi.at[b]).wait()
                ids_blk = ids[b].reshape(1, TB)
                onehot = (seg_iota == ids_blk).astype(jnp.float32)
                r = rows[b]
                r_hi = r.astype(jnp.bfloat16).astype(jnp.float32)
                r_cat = jnp.concatenate([r_hi, r - r_hi], axis=1)
                dims = (((1,), (0,)), ((), ()))
                prod = lax.dot_general(onehot, r_cat, dims,
                                       preferred_element_type=jnp.float32)
                acc_ref[...] += prod[:, :D] + prod[:, D:]
                cnt_ref[...] += jnp.sum(onehot, axis=1, keepdims=True)
        return carry
    lax.fori_loop(0, (nblk + NBUF - 1) // NBUF, _tstep, 0)


_tc_part = pl.pallas_call(
    _tc_body,
    grid_spec=pltpu.PrefetchScalarGridSpec(
        num_scalar_prefetch=1,
        grid=(N_TILES,),
        in_specs=[
            pl.BlockSpec(memory_space=pl.ANY),
            pl.BlockSpec(memory_space=pl.ANY),
        ],
        out_specs=[
            pl.BlockSpec((SEG_T, D), lambda t, b_ref: (t, 0)),
            pl.BlockSpec((SEG_T, 1), lambda t, b_ref: (t, 0)),
        ],
        scratch_shapes=[
            pltpu.VMEM((NBUF, TB, D), jnp.float32),
            pltpu.VMEM((NBUF, TB), jnp.int32),
            pltpu.SemaphoreType.DMA((NBUF,)),
            pltpu.SemaphoreType.DMA((NBUF,)),
        ],
    ),
    out_shape=[
        jax.ShapeDtypeStruct((S_PAD, D), jnp.float32),
        jax.ShapeDtypeStruct((S_PAD, 1), jnp.float32),
    ],
)


RB = 2000


def _merge_body(a_ref, c_ref, at_ref, ct_ref, o_ref):
    sums = a_ref[0] + a_ref[1] + at_ref[...]
    cnts = c_ref[0] + c_ref[1] + ct_ref[...]
    o_ref[...] = sums / jnp.maximum(cnts, 1.0)


_merge = pl.pallas_call(
    _merge_body,
    grid=(N_SEG // RB,),
    in_specs=[
        pl.BlockSpec((2, RB, D), lambda r: (0, r, 0)),
        pl.BlockSpec((2, RB, 1), lambda r: (0, r, 0)),
        pl.BlockSpec((RB, D), lambda r: (r, 0)),
        pl.BlockSpec((RB, 1), lambda r: (r, 0)),
    ],
    out_specs=pl.BlockSpec((RB, D), lambda r: (r, 0)),
    out_shape=jax.ShapeDtypeStruct((N_SEG, D), jnp.float32),
)


def kernel(input_values, segment_ids):
    ids32 = segment_ids.astype(jnp.int32)
    first = lax.slice(ids32, (H_SC,), (N_ROWS,), (TB,))
    nxt = jnp.concatenate(
        [first[1:], jnp.full((1,), jnp.iinfo(jnp.int32).max, jnp.int32)])
    tile_lo = jnp.arange(N_TILES, dtype=jnp.int32) * SEG_T
    blo = jnp.sum(nxt[None, :] < tile_lo[:, None], axis=1,
                  dtype=jnp.int32)
    bhi = jnp.sum(first[None, :] < (tile_lo[:, None] + SEG_T), axis=1,
                  dtype=jnp.int32)
    meta = jnp.concatenate([blo, jnp.maximum(bhi, blo)])
    acc_sc, cnt_sc = _sc_agg(input_values, ids32)
    acc_tc, cnt_tc = _tc_part(meta, input_values, ids32)
    return _merge(acc_sc, cnt_sc.reshape(2, S_PAD, 1), acc_tc, cnt_tc)
